# Initial kernel scaffold; baseline (speedup 1.0000x reference)
import jax
import jax.numpy as jnp
from jax.experimental import pallas as pl


def kernel(data):
    raise NotImplementedError("write your pallas kernel here")


# speedup vs baseline: 81.8695x; 81.8695x over previous
import functools
import math

import jax
import jax.numpy as jnp
from jax import lax
from jax.experimental import pallas as pl
from jax.experimental.pallas import tpu as pltpu
from jax.experimental.pallas import tpu_sc as plsc

N = 8192
D = 1024
K = 64
KMEANS_ITERS = 3
TOLERANCE = 1e-4
CAP = math.ceil(N / K)

BM = 1024
NB = N // BM


def _dist_body(data_ref, cbt_ref, asq_ref, bsq_ref, out_ref):
    a = data_ref[...]
    ab = jnp.dot(
        a.astype(jnp.bfloat16),
        cbt_ref[...].astype(jnp.bfloat16),
        preferred_element_type=jnp.float32,
    )
    sq = asq_ref[...] + bsq_ref[0:1, :] - 2.0 * ab
    out_ref[...] = jnp.sqrt(jnp.maximum(sq, 0.0) + 1e-9)


_dist_call = pl.pallas_call(
    _dist_body,
    grid=(NB,),
    in_specs=[
        pl.BlockSpec((BM, D), lambda i: (i, 0)),
        pl.BlockSpec((D, K), lambda i: (0, 0)),
        pl.BlockSpec((BM, 1), lambda i: (i, 0)),
        pl.BlockSpec((8, K), lambda i: (0, 0)),
    ],
    out_specs=pl.BlockSpec((BM, K), lambda i: (i, 0)),
    out_shape=jax.ShapeDtypeStruct((N, K), jnp.float32),
)

CHUNK = 512
NCH = N // CHUNK
GROUPS = CHUNK // 16

def _greedy_body(dist_hbm, out_hbm, buf, lab):
    cid = lax.axis_index("c")
    sid = lax.axis_index("s")

    @pl.when(jnp.logical_and(cid == 0, sid == 0))
    def _():
        lanes = lax.iota(jnp.int32, 16)
        big_f = jnp.float32(3.0e38)
        big_i = jnp.int32(1 << 20)
        perms = [lanes ^ s for s in (8, 4, 2, 1)]

        def _allmin(v):
            for p in perms:
                v = jnp.minimum(v, v.at[p].get(mode="promise_in_bounds"))
            return v

        def chunk_body(ci, counts):
            pltpu.sync_copy(dist_hbm.at[pl.ds(ci * (CHUNK * K), CHUNK * K)], buf)

            def group_body(g, counts2):
                c0, c1, c2, c3 = counts2

                def row_body(r, rc):
                    c0, c1, c2, c3, lv = rc
                    base = (g * 16 + r) * K
                    d0 = buf[pl.ds(base, 16)]
                    d1 = buf[pl.ds(base + 16, 16)]
                    d2 = buf[pl.ds(base + 32, 16)]
                    d3 = buf[pl.ds(base + 48, 16)]
                    v0 = jnp.where(c0 < CAP, d0, big_f)
                    v1 = jnp.where(c1 < CAP, d1, big_f)
                    v2 = jnp.where(c2 < CAP, d2, big_f)
                    v3 = jnp.where(c3 < CAP, d3, big_f)
                    m = jnp.minimum(jnp.minimum(v0, v1), jnp.minimum(v2, v3))
                    mn = _allmin(m)
                    cand0 = jnp.where(v0 == mn, lanes, big_i)
                    cand1 = jnp.where(v1 == mn, lanes + 16, big_i)
                    cand2 = jnp.where(v2 == mn, lanes + 32, big_i)
                    cand3 = jnp.where(v3 == mn, lanes + 48, big_i)
                    cv = jnp.minimum(
                        jnp.minimum(cand0, cand1), jnp.minimum(cand2, cand3)
                    )
                    label = _allmin(cv)
                    c0 = c0 + jnp.where(lanes == label, 1, 0)
                    c1 = c1 + jnp.where((lanes + 16) == label, 1, 0)
                    c2 = c2 + jnp.where((lanes + 32) == label, 1, 0)
                    c3 = c3 + jnp.where((lanes + 48) == label, 1, 0)
                    lv = jnp.where(lanes == r, label, lv)
                    return (c0, c1, c2, c3, lv)

                lv0 = jnp.zeros((16,), jnp.int32)
                c0, c1, c2, c3, lv = lax.fori_loop(
                    0, 16, row_body, (c0, c1, c2, c3, lv0)
                )
                lab[pl.ds(ci * CHUNK + g * 16, 16)] = lv
                return (c0, c1, c2, c3)

            return lax.fori_loop(0, GROUPS, group_body, counts)

        z = jnp.zeros((16,), jnp.int32)
        lax.fori_loop(0, NCH, chunk_body, (z, z, z, z))
        pltpu.sync_copy(lab, out_hbm)


@functools.cache
def _greedy_call():
    mesh = plsc.VectorSubcoreMesh(core_axis_name="c", subcore_axis_name="s")
    return pl.kernel(
        _greedy_body,
        out_type=jax.ShapeDtypeStruct((N,), jnp.int32),
        mesh=mesh,
        scratch_types=[
            pltpu.VMEM((CHUNK * K,), jnp.float32),
            pltpu.VMEM((N,), jnp.int32),
        ],
    )

BU = 1024
NBU = N // BU


def _update_body(data_ref, lab_ref, oldcb_ref, cb_out, normsq_out, sums):
    i = pl.program_id(0)

    @pl.when(i == 0)
    def _():
        sums[...] = jnp.zeros_like(sums)

    def row(r, carry):
        lbl = lab_ref[0, 0, r]
        sums[pl.ds(lbl, 1), :] += data_ref[pl.ds(r, 1), :]
        return carry

    lax.fori_loop(0, BU, row, jnp.int32(0))

    @pl.when(i == NBU - 1)
    def _():
        new_cb = sums[...] / jnp.float32(CAP)
        cb_out[...] = new_cb
        dcb = new_cb - oldcb_ref[...]
        normsq_out[0] = jnp.sum(dcb * dcb)


_update_call = pl.pallas_call(
    _update_body,
    grid=(NBU,),
    in_specs=[
        pl.BlockSpec((BU, D), lambda i: (i, 0)),
        pl.BlockSpec((1, 1, BU), lambda i: (i, 0, 0), memory_space=pltpu.SMEM),
        pl.BlockSpec((K, D), lambda i: (0, 0)),
    ],
    out_specs=[
        pl.BlockSpec((K, D), lambda i: (0, 0)),
        pl.BlockSpec(memory_space=pltpu.SMEM),
    ],
    out_shape=[
        jax.ShapeDtypeStruct((K, D), jnp.float32),
        jax.ShapeDtypeStruct((1,), jnp.float32),
    ],
    scratch_shapes=[
        pltpu.VMEM((K, D), jnp.float32),
    ],
)


def kernel(data):
    n, d = data.shape
    perm = jax.random.permutation(jax.random.key(42), n)
    cb = data[perm[:K], :]
    done = jnp.zeros((), jnp.bool_)
    labels = None
    for _ in range(KMEANS_ITERS):
        cbt = cb.T
        asq = jnp.sum(data * data, axis=-1)[:, None]
        bsq = jnp.tile(jnp.sum(cb * cb, axis=-1)[None, :], (8, 1))
        dist = _dist_call(data, cbt, asq, bsq)
        labels = _greedy_call()(dist.reshape(-1))
        new_cb, normsq = _update_call(data, labels.reshape(NBU, 1, BU), cb)
        converged = jnp.sqrt(normsq[0]) < TOLERANCE
        cb = jnp.where(done, cb, new_cb)
        done = jnp.logical_or(done, converged)
    return cb, labels
